# Initial kernel scaffold; baseline (speedup 1.0000x reference)
import jax
import jax.numpy as jnp
from jax.experimental import pallas as pl


def kernel(embed_user, embed_item, edge_weight, edge_index, users, pos_items, neg_items, epoch):
    raise NotImplementedError("write your pallas kernel here")


# speedup vs baseline: 1.0002x; 1.0002x over previous
import jax
import jax.numpy as jnp
from jax.experimental import pallas as pl
from jax.experimental.pallas import tpu as pltpu

N_USERS = 25000
N_ITEMS = 25000
EMB_DIM = 64
N_LAYERS = 3
BATCH = 4096
SSL_TEMP = 0.1
SSL_REG = 1e-06
ALPHA = 1.0
DECAY = 0.0001


def _normalize(x, eps=1e-12):
    n = jnp.linalg.norm(x, axis=-1, keepdims=True)
    return x / jnp.maximum(n, eps)


def _final_kernel(pos_s_ref, neg_s_ref, ssl_ref, reg_ref, mf_ref, ssl_o_ref, reg_o_ref):
    pos = pos_s_ref[...]
    neg = neg_s_ref[...]
    maxi = jnp.log(jax.nn.sigmoid(pos - neg) + 1e-06)
    mf_ref[...] = (-jnp.mean(maxi)).reshape(1, 1)
    ssl_o_ref[...] = ssl_ref[...]
    reg_o_ref[...] = reg_ref[...]


def kernel(embed_user, embed_item, edge_weight, edge_index, users, pos_items, neg_items, epoch):
    n_nodes = N_USERS + N_ITEMS
    all_emb = jnp.concatenate([embed_user, embed_item], axis=0)
    embs = [all_emb]
    src = edge_index[0]
    dst = edge_index[1]
    for _ in range(N_LAYERS):
        msg = all_emb[src] * edge_weight[:, None]
        all_emb = jax.ops.segment_sum(msg, dst, num_segments=n_nodes)
        embs.append(all_emb)
    light_out = (embs[0] + embs[1] + embs[2] + embs[3]) * 0.25
    all_users = light_out[:N_USERS]
    all_items = light_out[N_USERS:]

    users_emb = _normalize(all_users[users])
    pos_emb = _normalize(all_items[pos_items])
    neg_emb = _normalize(all_items[neg_items])
    pos_scores = jnp.sum(users_emb * pos_emb, axis=1)
    neg_scores = jnp.sum(users_emb * neg_emb, axis=1)

    context_emb = embs[2]
    c_u = context_emb[:N_USERS]
    c_i = context_emb[N_USERS:]
    n1 = _normalize(c_u[users])
    n2 = _normalize(embed_user[users])
    na = _normalize(embed_user)
    ttl_u = jnp.sum(jnp.exp(jnp.matmul(n1, na.T) / SSL_TEMP), axis=1)
    lu = -jnp.sum(jnp.sum(n1 * n2, axis=1) / SSL_TEMP - jnp.log(ttl_u))
    m1 = _normalize(c_i[pos_items])
    m2 = _normalize(embed_item[pos_items])
    ma = _normalize(embed_item)
    ttl_i = jnp.sum(jnp.exp(jnp.matmul(m1, ma.T) / SSL_TEMP), axis=1)
    li = -jnp.sum(jnp.sum(m1 * m2, axis=1) / SSL_TEMP - jnp.log(ttl_i))
    ssl_loss = SSL_REG * (lu + ALPHA * li)

    userEmb0 = embed_user[users]
    posEmb0 = embed_item[pos_items]
    negEmb0 = embed_item[neg_items]
    reg = 0.5 * (jnp.sum(userEmb0**2) + jnp.sum(posEmb0**2) + jnp.sum(negEmb0**2)) / BATCH
    reg_loss = DECAY * reg

    mf, ssl_o, reg_o = pl.pallas_call(
        _final_kernel,
        out_shape=[jax.ShapeDtypeStruct((1, 1), jnp.float32)] * 3,
    )(pos_scores.reshape(1, BATCH), neg_scores.reshape(1, BATCH),
      ssl_loss.reshape(1, 1), reg_loss.reshape(1, 1))
    return (mf[0, 0], ssl_o[0, 0], reg_o[0, 0])


# speedup vs baseline: 1.7649x; 1.7645x over previous
import functools

import jax
import jax.numpy as jnp
from jax import lax
from jax.experimental import pallas as pl
from jax.experimental.pallas import tpu as pltpu
from jax.experimental.pallas import tpu_sc as plsc

N_USERS = 25000
N_ITEMS = 25000
EMB_DIM = 64
N_LAYERS = 3
N_EDGES = 800000
BATCH = 4096
SSL_TEMP = 0.1
SSL_REG = 1e-06
ALPHA = 1.0
DECAY = 0.0001

N_NODES = N_USERS + N_ITEMS
NC, NS, LANES = 2, 16, 16
NW = NC * NS

HALF = N_NODES // 2
ACC_PAD = 88
ACC_ROWS = HALF + ACC_PAD
ZPT = ACC_ROWS // NS
TAIL = HALF - (NS - 1) * ZPT

EDGE_ROW = 128
ROWS_PER_TILE = 391
E_PAD = NS * ROWS_PER_TILE * EDGE_ROW

TBLK = 1600
TAB_PAD = 25600
N_TPAD = TAB_PAD - N_USERS


def _layer_body(emb, src2d, dst2d, w2d, out,
                src_v, dst_v, w_v, idx_v, wz_v, rows_v, acc_sh):
    c = lax.axis_index("c")
    s = lax.axis_index("s")
    base = c * HALF
    iota = lax.iota(jnp.int32, 16)
    zero16 = jnp.zeros((16,), jnp.float32)

    def zbody(i, _):
        for q in range(4):
            rows_v[i, pl.ds(q * 16, 16)] = zero16
        return 0
    lax.fori_loop(0, EDGE_ROW, zbody, 0)
    z0 = s * ZPT
    for z in range(12):
        pltpu.sync_copy(rows_v, acc_sh.at[pl.ds(z0 + z * 128, 128)])
    pltpu.sync_copy(rows_v.at[pl.ds(0, 32)], acc_sh.at[pl.ds(z0 + 1536, 32)])
    plsc.subcore_barrier()

    row0 = s * ROWS_PER_TILE

    def ebody(j, _):
        r = row0 + j
        pltpu.sync_copy(src2d.at[r], src_v)
        pltpu.sync_copy(dst2d.at[r], dst_v)
        pltpu.sync_copy(w2d.at[r], w_v)
        pltpu.sync_copy(emb.at[src_v], rows_v)
        for q in range(8):
            sl = pl.ds(q * 16, 16)
            d = dst_v[sl]
            wq = w_v[sl]
            inr = (d >= base) & (d < base + HALF)
            dummy = HALF + ((q * 16 + iota) % 64)
            idx_v[sl] = jnp.where(inr, d - base, dummy)
            wz_v[sl] = jnp.where(inr, wq, 0.0)

        def mbody(g, _):
            wz16 = wz_v[pl.ds(g * 16, 16)]
            for u in range(16):
                e = g * 16 + u
                wb = jnp.full((16,), wz16[u], jnp.float32)
                for q in range(4):
                    sl = pl.ds(q * 16, 16)
                    rows_v[e, sl] = rows_v[e, sl] * wb
            return 0
        lax.fori_loop(0, EDGE_ROW // 16, mbody, 0)
        pltpu.sync_copy(rows_v, acc_sh.at[idx_v], add=True)
        return 0
    lax.fori_loop(0, ROWS_PER_TILE, ebody, 0)
    plsc.subcore_barrier()

    @pl.when(s < NS - 1)
    def _():
        pltpu.sync_copy(acc_sh.at[pl.ds(z0, ZPT)],
                        out.at[pl.ds(base + z0, ZPT)])

    @pl.when(s == NS - 1)
    def _():
        pltpu.sync_copy(acc_sh.at[pl.ds(z0, TAIL)],
                        out.at[pl.ds(base + z0, TAIL)])


def _gather_body(eu, ei, e1, e2, e3, users, pos, neg,
                 lu_o, lp_o, ln_o, cu_o, ci_o, u0_o, p0_o, n0_o,
                 idx_v, gidx_v, acc_v, row_v):
    c = lax.axis_index("c")
    s = lax.axis_index("s")
    wid = s * NC + c
    nb = BATCH // NW
    b0 = wid * nb
    sl_out = pl.ds(b0, nb)

    def shift(dst_ref, src_ref):
        for q in range(nb // 16):
            sl = pl.ds(q * 16, 16)
            dst_ref[sl] = src_ref[sl] + N_USERS

    pltpu.sync_copy(users.at[sl_out], idx_v)
    pltpu.sync_copy(eu.at[idx_v], acc_v)
    pltpu.sync_copy(e1.at[idx_v], acc_v, add=True)
    pltpu.sync_copy(e2.at[idx_v], acc_v, add=True)
    pltpu.sync_copy(e3.at[idx_v], acc_v, add=True)
    pltpu.sync_copy(acc_v, lu_o.at[sl_out])
    pltpu.sync_copy(e2.at[idx_v], row_v)
    pltpu.sync_copy(row_v, cu_o.at[sl_out])
    pltpu.sync_copy(eu.at[idx_v], row_v)
    pltpu.sync_copy(row_v, u0_o.at[sl_out])
    pltpu.sync_copy(pos.at[sl_out], idx_v)
    shift(gidx_v, idx_v)
    pltpu.sync_copy(ei.at[idx_v], acc_v)
    pltpu.sync_copy(e1.at[gidx_v], acc_v, add=True)
    pltpu.sync_copy(e2.at[gidx_v], acc_v, add=True)
    pltpu.sync_copy(e3.at[gidx_v], acc_v, add=True)
    pltpu.sync_copy(acc_v, lp_o.at[sl_out])
    pltpu.sync_copy(e2.at[gidx_v], row_v)
    pltpu.sync_copy(row_v, ci_o.at[sl_out])
    pltpu.sync_copy(ei.at[idx_v], row_v)
    pltpu.sync_copy(row_v, p0_o.at[sl_out])
    pltpu.sync_copy(neg.at[sl_out], idx_v)
    shift(gidx_v, idx_v)
    pltpu.sync_copy(ei.at[idx_v], acc_v)
    pltpu.sync_copy(e1.at[gidx_v], acc_v, add=True)
    pltpu.sync_copy(e2.at[gidx_v], acc_v, add=True)
    pltpu.sync_copy(e3.at[gidx_v], acc_v, add=True)
    pltpu.sync_copy(acc_v, ln_o.at[sl_out])
    pltpu.sync_copy(ei.at[idx_v], row_v)
    pltpu.sync_copy(row_v, n0_o.at[sl_out])


def _norm_rows(x, eps=1e-12):
    n = jnp.sqrt(jnp.sum(x * x, axis=-1, keepdims=True))
    return x / jnp.maximum(n, eps)


def _ttl_body(tab_ref, c_ref, ou_ref, oi_ref):
    t = pl.program_id(0)
    b = pl.program_id(1)
    n1 = _norm_rows(c_ref[0])
    na = _norm_rows(tab_ref[0])
    sc = lax.dot_general(n1, na, (((1,), (1,)), ((), ())),
                         preferred_element_type=jnp.float32)
    r = jnp.sum(jnp.exp(sc * (1.0 / SSL_TEMP)), axis=1).reshape(1, BATCH)

    @pl.when((t == 0) & (b == 0))
    def _():
        ou_ref[...] = r

    @pl.when((t == 0) & (b > 0))
    def _():
        ou_ref[...] = ou_ref[...] + r

    @pl.when((t == 1) & (b == 0))
    def _():
        oi_ref[...] = r

    @pl.when((t == 1) & (b > 0))
    def _():
        oi_ref[...] = oi_ref[...] + r


def _final_body(ttl_ref, cu_ref, ci_ref, u0_ref, p0_ref, n0_ref,
                lu_ref, lp_ref, ln_ref, mf_o, ssl_o, reg_o):
    cu = cu_ref[...]
    u0 = u0_ref[...]
    ci = ci_ref[...]
    p0 = p0_ref[...]
    n0 = n0_ref[...]
    du = jnp.sum(_norm_rows(cu) * _norm_rows(u0), axis=1)
    di = jnp.sum(_norm_rows(ci) * _norm_rows(p0), axis=1)
    ttl_u = ttl_ref[0, :] - float(N_TPAD)
    ttl_i = ttl_ref[1, :] - float(N_TPAD)
    lu_loss = -jnp.sum(du * (1.0 / SSL_TEMP) - jnp.log(ttl_u))
    li_loss = -jnp.sum(di * (1.0 / SSL_TEMP) - jnp.log(ttl_i))
    ssl = SSL_REG * (lu_loss + ALPHA * li_loss)
    ue = _norm_rows(lu_ref[...])
    pe = _norm_rows(lp_ref[...])
    ne = _norm_rows(ln_ref[...])
    x = jnp.sum(ue * pe, axis=1) - jnp.sum(ue * ne, axis=1)
    sig = 1.0 / (1.0 + jnp.exp(-x))
    mf = -jnp.mean(jnp.log(sig + 1e-6))
    reg = DECAY * 0.5 * (jnp.sum(u0 * u0) + jnp.sum(p0 * p0)
                         + jnp.sum(n0 * n0)) / BATCH
    mf_o[...] = jnp.reshape(mf, (1, 1))
    ssl_o[...] = jnp.reshape(ssl, (1, 1))
    reg_o[...] = jnp.reshape(reg, (1, 1))


def _sc_mesh():
    return plsc.VectorSubcoreMesh(core_axis_name="c", subcore_axis_name="s",
                                  num_cores=NC, num_subcores=NS)


_SC_PARAMS = pltpu.CompilerParams(use_tc_tiling_on_sc=False)


def _make_layer_call():
    return pl.kernel(
        _layer_body,
        out_type=jax.ShapeDtypeStruct((N_NODES, EMB_DIM), jnp.float32),
        mesh=_sc_mesh(),
        compiler_params=_SC_PARAMS,
        scratch_types=[
            pltpu.VMEM((EDGE_ROW,), jnp.int32),
            pltpu.VMEM((EDGE_ROW,), jnp.int32),
            pltpu.VMEM((EDGE_ROW,), jnp.float32),
            pltpu.VMEM((EDGE_ROW,), jnp.int32),
            pltpu.VMEM((EDGE_ROW,), jnp.float32),
            pltpu.VMEM((EDGE_ROW, EMB_DIM), jnp.float32),
            pltpu.VMEM_SHARED((ACC_ROWS, EMB_DIM), jnp.float32),
        ],
    )


def _make_gather_call():
    shp = jax.ShapeDtypeStruct((BATCH, EMB_DIM), jnp.float32)
    nb = BATCH // NW
    return pl.kernel(
        _gather_body,
        out_type=[shp] * 8,
        mesh=_sc_mesh(),
        compiler_params=_SC_PARAMS,
        scratch_types=[
            pltpu.VMEM((nb,), jnp.int32),
            pltpu.VMEM((nb,), jnp.int32),
            pltpu.VMEM((nb, EMB_DIM), jnp.float32),
            pltpu.VMEM((nb, EMB_DIM), jnp.float32),
        ],
    )


def kernel(embed_user, embed_item, edge_weight, edge_index, users, pos_items,
           neg_items, epoch):
    f32 = jnp.float32
    src = edge_index[0]
    dst = edge_index[1]
    pad = E_PAD - N_EDGES
    src2d = jnp.concatenate([src, jnp.zeros((pad,), src.dtype)]).reshape(-1, EDGE_ROW)
    dst2d = jnp.concatenate([dst, jnp.zeros((pad,), dst.dtype)]).reshape(-1, EDGE_ROW)
    w2d = jnp.concatenate([edge_weight, jnp.zeros((pad,), f32)]).reshape(-1, EDGE_ROW)

    all0 = jnp.concatenate([embed_user, embed_item], axis=0)
    layer = _make_layer_call()
    e1 = layer(all0, src2d, dst2d, w2d)
    e2 = layer(e1, src2d, dst2d, w2d)
    e3 = layer(e2, src2d, dst2d, w2d)

    gather = _make_gather_call()
    lsum_u, lsum_p, lsum_n, cu, ci, u0, p0, n0 = gather(
        embed_user, embed_item, e1, e2, e3, users, pos_items, neg_items)

    zpad = jnp.zeros((N_TPAD, EMB_DIM), f32)
    tabs = jnp.stack([jnp.concatenate([embed_user, zpad], axis=0),
                      jnp.concatenate([embed_item, zpad], axis=0)])
    cstack = jnp.stack([cu, ci])
    nblk = TAB_PAD // TBLK
    ttl_u, ttl_i = pl.pallas_call(
        _ttl_body,
        grid=(2, nblk),
        in_specs=[pl.BlockSpec((1, TBLK, EMB_DIM), lambda t, b: (t, b, 0)),
                  pl.BlockSpec((1, BATCH, EMB_DIM), lambda t, b: (t, 0, 0))],
        out_specs=[pl.BlockSpec((1, BATCH), lambda t, b: (0, 0))] * 2,
        out_shape=[jax.ShapeDtypeStruct((1, BATCH), f32)] * 2,
    )(tabs, cstack)
    ttl = jnp.concatenate([ttl_u, ttl_i], axis=0)

    mf, ssl, reg = pl.pallas_call(
        _final_body,
        out_shape=[jax.ShapeDtypeStruct((1, 1), f32)] * 3,
    )(ttl, cu, ci, u0, p0, n0, lsum_u, lsum_p, lsum_n)
    return (mf[0, 0], ssl[0, 0], reg[0, 0])


# speedup vs baseline: 3.0282x; 1.7158x over previous
import functools

import jax
import jax.numpy as jnp
from jax import lax
from jax.experimental import pallas as pl
from jax.experimental.pallas import tpu as pltpu
from jax.experimental.pallas import tpu_sc as plsc

N_USERS = 25000
N_ITEMS = 25000
EMB_DIM = 64
N_LAYERS = 3
N_EDGES = 800000
BATCH = 4096
SSL_TEMP = 0.1
SSL_REG = 1e-06
ALPHA = 1.0
DECAY = 0.0001

N_NODES = N_USERS + N_ITEMS
NC, NS, LANES = 2, 16, 16
NW = NC * NS

HALF = N_NODES // 2
ACC_PAD = 88
ACC_ROWS = HALF + ACC_PAD
ZPT = ACC_ROWS // NS
TAIL = HALF - (NS - 1) * ZPT

EDGE_ROW = 128
ROWS_PER_TILE = 392
E_PAD = NS * ROWS_PER_TILE * EDGE_ROW

TBLK = 1600
TAB_PAD = 25600
N_TPAD = TAB_PAD - N_USERS


def _layer_body(emb, ed3, out,
                eb0, eb1, idx0, idx1, wz0, wz1, rows0, rows1, acc_sh,
                esem0, esem1, gsem0, gsem1, ssem0, ssem1):
    c = lax.axis_index("c")
    s = lax.axis_index("s")
    base = c * HALF
    iota = lax.iota(jnp.int32, 16)
    zero16 = jnp.zeros((16,), jnp.float32)
    ebs = (eb0, eb1)
    idxs = (idx0, idx1)
    wzs = (wz0, wz1)
    rows = (rows0, rows1)
    esems = (esem0, esem1)
    gsems = (gsem0, gsem1)
    ssems = (ssem0, ssem1)

    def zbody(i, _):
        for q in range(4):
            rows0[i, pl.ds(q * 16, 16)] = zero16
        return 0
    lax.fori_loop(0, EDGE_ROW, zbody, 0)
    z0 = s * ZPT
    for z in range(12):
        pltpu.sync_copy(rows0, acc_sh.at[pl.ds(z0 + z * 128, 128)])
    pltpu.sync_copy(rows0.at[pl.ds(0, 32)], acc_sh.at[pl.ds(z0 + 1536, 32)])
    plsc.subcore_barrier()

    row0 = s * ROWS_PER_TILE

    pltpu.async_copy(ed3.at[row0], eb0, esem0)
    pltpu.async_copy(ed3.at[row0 + 1], eb1, esem1)
    pltpu.make_async_copy(ed3.at[row0], eb0, esem0).wait()
    pltpu.async_copy(emb.at[eb0.at[0]], rows0, gsem0)

    def pair(step, _):
        for b in range(2):
            nb = 1 - b
            i = step * 2 + b
            j = row0 + i
            for q in range(8):
                sl = pl.ds(q * 16, 16)
                d = ebs[b][1, sl]
                wq = plsc.bitcast(ebs[b][2, sl], jnp.float32)
                inr = (d >= base) & (d < base + HALF)
                dummy = HALF + ((q * 16 + iota) % 64)
                idxs[b][sl] = jnp.where(inr, d - base, dummy)
                wzs[b][sl] = jnp.where(inr, wq, 0.0)

            @pl.when(i < ROWS_PER_TILE - 1)
            def _():
                pltpu.make_async_copy(ed3.at[j + 1], ebs[nb], esems[nb]).wait()

                @pl.when(i >= 1)
                def _():
                    pltpu.make_async_copy(
                        rows[nb], acc_sh.at[idxs[nb]], ssems[nb]).wait()
                pltpu.async_copy(emb.at[ebs[nb].at[0]], rows[nb], gsems[nb])

            pltpu.make_async_copy(emb.at[ebs[b].at[0]], rows[b], gsems[b]).wait()

            @pl.when(i < ROWS_PER_TILE - 2)
            def _():
                pltpu.async_copy(ed3.at[j + 2], ebs[b], esems[b])

            def mbody(g, _):
                wz16 = wzs[b][pl.ds(g * 16, 16)]
                for u in range(16):
                    e = g * 16 + u
                    wbv = jnp.full((16,), wz16[u], jnp.float32)
                    for q in range(4):
                        sl = pl.ds(q * 16, 16)
                        rows[b][e, sl] = rows[b][e, sl] * wbv
                return 0
            lax.fori_loop(0, EDGE_ROW // 16, mbody, 0)
            pltpu.async_copy(rows[b], acc_sh.at[idxs[b]], ssems[b], add=True)
        return 0
    lax.fori_loop(0, ROWS_PER_TILE // 2, pair, 0)
    pltpu.make_async_copy(rows0, acc_sh.at[idx0], ssem0).wait()
    pltpu.make_async_copy(rows1, acc_sh.at[idx1], ssem1).wait()
    plsc.subcore_barrier()

    @pl.when(s < NS - 1)
    def _():
        pltpu.sync_copy(acc_sh.at[pl.ds(z0, ZPT)],
                        out.at[pl.ds(base + z0, ZPT)])

    @pl.when(s == NS - 1)
    def _():
        pltpu.sync_copy(acc_sh.at[pl.ds(z0, TAIL)],
                        out.at[pl.ds(base + z0, TAIL)])


def _gather_body(eu, ei, e1, e2, e3, users, pos, neg,
                 lu_o, lp_o, ln_o, cu_o, ci_o, u0_o, p0_o, n0_o,
                 idx_v, gidx_v, acc_v, row_v):
    c = lax.axis_index("c")
    s = lax.axis_index("s")
    wid = s * NC + c
    nb = BATCH // NW
    b0 = wid * nb
    sl_out = pl.ds(b0, nb)

    def shift(dst_ref, src_ref):
        for q in range(nb // 16):
            sl = pl.ds(q * 16, 16)
            dst_ref[sl] = src_ref[sl] + N_USERS

    pltpu.sync_copy(users.at[sl_out], idx_v)
    pltpu.sync_copy(eu.at[idx_v], acc_v)
    pltpu.sync_copy(e1.at[idx_v], acc_v, add=True)
    pltpu.sync_copy(e2.at[idx_v], acc_v, add=True)
    pltpu.sync_copy(e3.at[idx_v], acc_v, add=True)
    pltpu.sync_copy(acc_v, lu_o.at[sl_out])
    pltpu.sync_copy(e2.at[idx_v], row_v)
    pltpu.sync_copy(row_v, cu_o.at[sl_out])
    pltpu.sync_copy(eu.at[idx_v], row_v)
    pltpu.sync_copy(row_v, u0_o.at[sl_out])
    pltpu.sync_copy(pos.at[sl_out], idx_v)
    shift(gidx_v, idx_v)
    pltpu.sync_copy(ei.at[idx_v], acc_v)
    pltpu.sync_copy(e1.at[gidx_v], acc_v, add=True)
    pltpu.sync_copy(e2.at[gidx_v], acc_v, add=True)
    pltpu.sync_copy(e3.at[gidx_v], acc_v, add=True)
    pltpu.sync_copy(acc_v, lp_o.at[sl_out])
    pltpu.sync_copy(e2.at[gidx_v], row_v)
    pltpu.sync_copy(row_v, ci_o.at[sl_out])
    pltpu.sync_copy(ei.at[idx_v], row_v)
    pltpu.sync_copy(row_v, p0_o.at[sl_out])
    pltpu.sync_copy(neg.at[sl_out], idx_v)
    shift(gidx_v, idx_v)
    pltpu.sync_copy(ei.at[idx_v], acc_v)
    pltpu.sync_copy(e1.at[gidx_v], acc_v, add=True)
    pltpu.sync_copy(e2.at[gidx_v], acc_v, add=True)
    pltpu.sync_copy(e3.at[gidx_v], acc_v, add=True)
    pltpu.sync_copy(acc_v, ln_o.at[sl_out])
    pltpu.sync_copy(ei.at[idx_v], row_v)
    pltpu.sync_copy(row_v, n0_o.at[sl_out])


def _norm_rows(x, eps=1e-12):
    n = jnp.sqrt(jnp.sum(x * x, axis=-1, keepdims=True))
    return x / jnp.maximum(n, eps)


def _ttl_body(tab_ref, c_ref, ou_ref, oi_ref):
    t = pl.program_id(0)
    b = pl.program_id(1)
    n1 = _norm_rows(c_ref[0])
    na = _norm_rows(tab_ref[0])
    sc = lax.dot_general(n1, na, (((1,), (1,)), ((), ())),
                         preferred_element_type=jnp.float32)
    r = jnp.sum(jnp.exp(sc * (1.0 / SSL_TEMP)), axis=1).reshape(1, BATCH)

    @pl.when((t == 0) & (b == 0))
    def _():
        ou_ref[...] = r

    @pl.when((t == 0) & (b > 0))
    def _():
        ou_ref[...] = ou_ref[...] + r

    @pl.when((t == 1) & (b == 0))
    def _():
        oi_ref[...] = r

    @pl.when((t == 1) & (b > 0))
    def _():
        oi_ref[...] = oi_ref[...] + r


def _final_body(ttl_ref, cu_ref, ci_ref, u0_ref, p0_ref, n0_ref,
                lu_ref, lp_ref, ln_ref, mf_o, ssl_o, reg_o):
    cu = cu_ref[...]
    u0 = u0_ref[...]
    ci = ci_ref[...]
    p0 = p0_ref[...]
    n0 = n0_ref[...]
    du = jnp.sum(_norm_rows(cu) * _norm_rows(u0), axis=1)
    di = jnp.sum(_norm_rows(ci) * _norm_rows(p0), axis=1)
    ttl_u = ttl_ref[0, :] - float(N_TPAD)
    ttl_i = ttl_ref[1, :] - float(N_TPAD)
    lu_loss = -jnp.sum(du * (1.0 / SSL_TEMP) - jnp.log(ttl_u))
    li_loss = -jnp.sum(di * (1.0 / SSL_TEMP) - jnp.log(ttl_i))
    ssl = SSL_REG * (lu_loss + ALPHA * li_loss)
    ue = _norm_rows(lu_ref[...])
    pe = _norm_rows(lp_ref[...])
    ne = _norm_rows(ln_ref[...])
    x = jnp.sum(ue * pe, axis=1) - jnp.sum(ue * ne, axis=1)
    sig = 1.0 / (1.0 + jnp.exp(-x))
    mf = -jnp.mean(jnp.log(sig + 1e-6))
    reg = DECAY * 0.5 * (jnp.sum(u0 * u0) + jnp.sum(p0 * p0)
                         + jnp.sum(n0 * n0)) / BATCH
    mf_o[...] = jnp.reshape(mf, (1, 1))
    ssl_o[...] = jnp.reshape(ssl, (1, 1))
    reg_o[...] = jnp.reshape(reg, (1, 1))


def _sc_mesh():
    return plsc.VectorSubcoreMesh(core_axis_name="c", subcore_axis_name="s",
                                  num_cores=NC, num_subcores=NS)


_SC_PARAMS = pltpu.CompilerParams(use_tc_tiling_on_sc=False,
                                  needs_layout_passes=False)


def _make_layer_call():
    return pl.kernel(
        _layer_body,
        out_type=jax.ShapeDtypeStruct((N_NODES, EMB_DIM), jnp.float32),
        mesh=_sc_mesh(),
        compiler_params=_SC_PARAMS,
        scratch_types=[
            pltpu.VMEM((3, EDGE_ROW), jnp.int32),
            pltpu.VMEM((3, EDGE_ROW), jnp.int32),
            pltpu.VMEM((EDGE_ROW,), jnp.int32),
            pltpu.VMEM((EDGE_ROW,), jnp.int32),
            pltpu.VMEM((EDGE_ROW,), jnp.float32),
            pltpu.VMEM((EDGE_ROW,), jnp.float32),
            pltpu.VMEM((EDGE_ROW, EMB_DIM), jnp.float32),
            pltpu.VMEM((EDGE_ROW, EMB_DIM), jnp.float32),
            pltpu.VMEM_SHARED((ACC_ROWS, EMB_DIM), jnp.float32),
            pltpu.SemaphoreType.DMA,
            pltpu.SemaphoreType.DMA,
            pltpu.SemaphoreType.DMA,
            pltpu.SemaphoreType.DMA,
            pltpu.SemaphoreType.DMA,
            pltpu.SemaphoreType.DMA,
        ],
    )


def _make_gather_call():
    shp = jax.ShapeDtypeStruct((BATCH, EMB_DIM), jnp.float32)
    nb = BATCH // NW
    return pl.kernel(
        _gather_body,
        out_type=[shp] * 8,
        mesh=_sc_mesh(),
        compiler_params=_SC_PARAMS,
        scratch_types=[
            pltpu.VMEM((nb,), jnp.int32),
            pltpu.VMEM((nb,), jnp.int32),
            pltpu.VMEM((nb, EMB_DIM), jnp.float32),
            pltpu.VMEM((nb, EMB_DIM), jnp.float32),
        ],
    )


def kernel(embed_user, embed_item, edge_weight, edge_index, users, pos_items,
           neg_items, epoch):
    f32 = jnp.float32
    src = edge_index[0]
    dst = edge_index[1]
    pad = E_PAD - N_EDGES
    src2d = jnp.concatenate([src, jnp.zeros((pad,), src.dtype)]).reshape(-1, EDGE_ROW)
    dst2d = jnp.concatenate([dst, jnp.zeros((pad,), dst.dtype)]).reshape(-1, EDGE_ROW)
    wbits = lax.bitcast_convert_type(
        jnp.concatenate([edge_weight, jnp.zeros((pad,), f32)]), jnp.int32
    ).reshape(-1, EDGE_ROW)
    ed3 = jnp.stack([src2d, dst2d, wbits], axis=1)

    all0 = jnp.concatenate([embed_user, embed_item], axis=0)
    layer = _make_layer_call()
    e1 = layer(all0, ed3)
    e2 = layer(e1, ed3)
    e3 = layer(e2, ed3)

    gather = _make_gather_call()
    lsum_u, lsum_p, lsum_n, cu, ci, u0, p0, n0 = gather(
        embed_user, embed_item, e1, e2, e3, users, pos_items, neg_items)

    zpad = jnp.zeros((N_TPAD, EMB_DIM), f32)
    tabs = jnp.stack([jnp.concatenate([embed_user, zpad], axis=0),
                      jnp.concatenate([embed_item, zpad], axis=0)])
    cstack = jnp.stack([cu, ci])
    nblk = TAB_PAD // TBLK
    ttl_u, ttl_i = pl.pallas_call(
        _ttl_body,
        grid=(2, nblk),
        in_specs=[pl.BlockSpec((1, TBLK, EMB_DIM), lambda t, b: (t, b, 0)),
                  pl.BlockSpec((1, BATCH, EMB_DIM), lambda t, b: (t, 0, 0))],
        out_specs=[pl.BlockSpec((1, BATCH), lambda t, b: (0, 0))] * 2,
        out_shape=[jax.ShapeDtypeStruct((1, BATCH), f32)] * 2,
    )(tabs, cstack)
    ttl = jnp.concatenate([ttl_u, ttl_i], axis=0)

    mf, ssl, reg = pl.pallas_call(
        _final_body,
        out_shape=[jax.ShapeDtypeStruct((1, 1), f32)] * 3,
    )(ttl, cu, ci, u0, p0, n0, lsum_u, lsum_p, lsum_n)
    return (mf[0, 0], ssl[0, 0], reg[0, 0])


# speedup vs baseline: 4.7835x; 1.5796x over previous
import functools

import jax
import jax.numpy as jnp
from jax import lax
from jax.experimental import pallas as pl
from jax.experimental.pallas import tpu as pltpu
from jax.experimental.pallas import tpu_sc as plsc

N_USERS = 25000
N_ITEMS = 25000
EMB_DIM = 64
N_LAYERS = 3
N_EDGES = 800000
BATCH = 4096
SSL_TEMP = 0.1
SSL_REG = 1e-06
ALPHA = 1.0
DECAY = 0.0001

N_NODES = N_USERS + N_ITEMS
NC, NS, LANES = 2, 16, 16
NW = NC * NS

HALF = N_NODES // 2
ACC_PAD = 88
ACC_ROWS = HALF + ACC_PAD
ZPT = ACC_ROWS // NS
TAIL = HALF - (NS - 1) * ZPT

EDGE_ROW = 64
ROWS_PER_TILE = 784
E_PAD = NS * ROWS_PER_TILE * EDGE_ROW

TBLK = 1600
TAB_PAD = 25600
N_TPAD = TAB_PAD - N_USERS


def _layer_body(emb, ed3, out,
                eb0, eb1, idx0, idx1, wz0, wz1, rows0, rows1, prod0, prod1,
                acc_sh, esem0, esem1, gsem0, gsem1, ssem0, ssem1):
    c = lax.axis_index("c")
    s = lax.axis_index("s")
    base = c * HALF
    iota = lax.iota(jnp.int32, 16)
    zero16 = jnp.zeros((16,), jnp.float32)
    ebs = (eb0, eb1)
    idxs = (idx0, idx1)
    wzs = (wz0, wz1)
    rows = (rows0, rows1)
    prods = (prod0, prod1)
    esems = (esem0, esem1)
    gsems = (gsem0, gsem1)
    ssems = (ssem0, ssem1)

    def zbody(i, _):
        for q in range(4):
            rows0[i, pl.ds(q * 16, 16)] = zero16
        return 0
    lax.fori_loop(0, EDGE_ROW, zbody, 0)
    z0 = s * ZPT
    for z in range(ZPT // EDGE_ROW):
        pltpu.sync_copy(rows0,
                        acc_sh.at[pl.ds(z0 + z * EDGE_ROW, EDGE_ROW)])
    pltpu.sync_copy(rows0.at[pl.ds(0, ZPT % EDGE_ROW)],
                    acc_sh.at[pl.ds(z0 + ZPT - ZPT % EDGE_ROW, ZPT % EDGE_ROW)])
    plsc.subcore_barrier()

    row0 = s * ROWS_PER_TILE

    pltpu.async_copy(ed3.at[row0], eb0, esem0)
    pltpu.async_copy(ed3.at[row0 + 1], eb1, esem1)
    pltpu.make_async_copy(ed3.at[row0], eb0, esem0).wait()
    pltpu.async_copy(emb.at[eb0.at[0]], rows0, gsem0)

    def pair(step, _):
        for b in range(2):
            nb = 1 - b
            i = step * 2 + b
            j = row0 + i
            for q in range(EDGE_ROW // 16):
                sl = pl.ds(q * 16, 16)
                d = ebs[b][1, sl]
                wq = plsc.bitcast(ebs[b][2, sl], jnp.float32)
                inr = (d >= base) & (d < base + HALF)
                dummy = HALF + ((q * 16 + iota) % 64)
                idxs[b][sl] = jnp.where(inr, d - base, dummy)
                wzs[b][sl] = jnp.where(inr, wq, 0.0)

            @pl.when(i < ROWS_PER_TILE - 1)
            def _():
                pltpu.make_async_copy(ed3.at[j + 1], ebs[nb], esems[nb]).wait()
                pltpu.async_copy(emb.at[ebs[nb].at[0]], rows[nb], gsems[nb])

            pltpu.make_async_copy(emb.at[ebs[b].at[0]], rows[b], gsems[b]).wait()

            @pl.when(i < ROWS_PER_TILE - 2)
            def _():
                pltpu.async_copy(ed3.at[j + 2], ebs[b], esems[b])

            @pl.when(i >= 2)
            def _():
                pltpu.make_async_copy(
                    prods[b], acc_sh.at[idxs[b]], ssems[b]).wait()

            def mbody(g, _):
                wz16 = wzs[b][pl.ds(g * 16, 16)]
                for u in range(16):
                    e = g * 16 + u
                    wbv = jnp.full((16,), wz16[u], jnp.float32)
                    r0 = rows[b][e, pl.ds(0, 16)]
                    r1 = rows[b][e, pl.ds(16, 16)]
                    r2 = rows[b][e, pl.ds(32, 16)]
                    r3 = rows[b][e, pl.ds(48, 16)]
                    prods[b][e, pl.ds(0, 16)] = r0 * wbv
                    prods[b][e, pl.ds(16, 16)] = r1 * wbv
                    prods[b][e, pl.ds(32, 16)] = r2 * wbv
                    prods[b][e, pl.ds(48, 16)] = r3 * wbv
                return 0
            lax.fori_loop(0, EDGE_ROW // 16, mbody, 0)
            pltpu.async_copy(prods[b], acc_sh.at[idxs[b]], ssems[b], add=True)
        return 0
    lax.fori_loop(0, ROWS_PER_TILE // 2, pair, 0)
    pltpu.make_async_copy(prod0, acc_sh.at[idx0], ssem0).wait()
    pltpu.make_async_copy(prod1, acc_sh.at[idx1], ssem1).wait()
    plsc.subcore_barrier()

    @pl.when(s < NS - 1)
    def _():
        pltpu.sync_copy(acc_sh.at[pl.ds(z0, ZPT)],
                        out.at[pl.ds(base + z0, ZPT)])

    @pl.when(s == NS - 1)
    def _():
        pltpu.sync_copy(acc_sh.at[pl.ds(z0, TAIL)],
                        out.at[pl.ds(base + z0, TAIL)])


def _gather_body(eu, ei, e1, e2, e3, users, pos, neg,
                 lu_o, lp_o, ln_o, cu_o, ci_o, u0_o, p0_o, n0_o,
                 idx_v, gidx_v, acc_v, row_v):
    c = lax.axis_index("c")
    s = lax.axis_index("s")
    wid = s * NC + c
    nb = BATCH // NW
    b0 = wid * nb
    sl_out = pl.ds(b0, nb)

    def shift(dst_ref, src_ref):
        for q in range(nb // 16):
            sl = pl.ds(q * 16, 16)
            dst_ref[sl] = src_ref[sl] + N_USERS

    pltpu.sync_copy(users.at[sl_out], idx_v)
    pltpu.sync_copy(eu.at[idx_v], acc_v)
    pltpu.sync_copy(e1.at[idx_v], acc_v, add=True)
    pltpu.sync_copy(e2.at[idx_v], acc_v, add=True)
    pltpu.sync_copy(e3.at[idx_v], acc_v, add=True)
    pltpu.sync_copy(acc_v, lu_o.at[sl_out])
    pltpu.sync_copy(e2.at[idx_v], row_v)
    pltpu.sync_copy(row_v, cu_o.at[sl_out])
    pltpu.sync_copy(eu.at[idx_v], row_v)
    pltpu.sync_copy(row_v, u0_o.at[sl_out])
    pltpu.sync_copy(pos.at[sl_out], idx_v)
    shift(gidx_v, idx_v)
    pltpu.sync_copy(ei.at[idx_v], acc_v)
    pltpu.sync_copy(e1.at[gidx_v], acc_v, add=True)
    pltpu.sync_copy(e2.at[gidx_v], acc_v, add=True)
    pltpu.sync_copy(e3.at[gidx_v], acc_v, add=True)
    pltpu.sync_copy(acc_v, lp_o.at[sl_out])
    pltpu.sync_copy(e2.at[gidx_v], row_v)
    pltpu.sync_copy(row_v, ci_o.at[sl_out])
    pltpu.sync_copy(ei.at[idx_v], row_v)
    pltpu.sync_copy(row_v, p0_o.at[sl_out])
    pltpu.sync_copy(neg.at[sl_out], idx_v)
    shift(gidx_v, idx_v)
    pltpu.sync_copy(ei.at[idx_v], acc_v)
    pltpu.sync_copy(e1.at[gidx_v], acc_v, add=True)
    pltpu.sync_copy(e2.at[gidx_v], acc_v, add=True)
    pltpu.sync_copy(e3.at[gidx_v], acc_v, add=True)
    pltpu.sync_copy(acc_v, ln_o.at[sl_out])
    pltpu.sync_copy(ei.at[idx_v], row_v)
    pltpu.sync_copy(row_v, n0_o.at[sl_out])


def _norm_rows(x, eps=1e-12):
    n = jnp.sqrt(jnp.sum(x * x, axis=-1, keepdims=True))
    return x / jnp.maximum(n, eps)


def _ttl_body(tab_ref, c_ref, ou_ref, oi_ref):
    t = pl.program_id(0)
    b = pl.program_id(1)
    n1 = _norm_rows(c_ref[0])
    na = _norm_rows(tab_ref[0])
    sc = lax.dot_general(n1, na, (((1,), (1,)), ((), ())),
                         preferred_element_type=jnp.float32)
    r = jnp.sum(jnp.exp(sc * (1.0 / SSL_TEMP)), axis=1).reshape(1, BATCH)

    @pl.when((t == 0) & (b == 0))
    def _():
        ou_ref[...] = r

    @pl.when((t == 0) & (b > 0))
    def _():
        ou_ref[...] = ou_ref[...] + r

    @pl.when((t == 1) & (b == 0))
    def _():
        oi_ref[...] = r

    @pl.when((t == 1) & (b > 0))
    def _():
        oi_ref[...] = oi_ref[...] + r


def _final_body(ttl_ref, cu_ref, ci_ref, u0_ref, p0_ref, n0_ref,
                lu_ref, lp_ref, ln_ref, mf_o, ssl_o, reg_o):
    cu = cu_ref[...]
    u0 = u0_ref[...]
    ci = ci_ref[...]
    p0 = p0_ref[...]
    n0 = n0_ref[...]
    du = jnp.sum(_norm_rows(cu) * _norm_rows(u0), axis=1)
    di = jnp.sum(_norm_rows(ci) * _norm_rows(p0), axis=1)
    ttl_u = ttl_ref[0, :] - float(N_TPAD)
    ttl_i = ttl_ref[1, :] - float(N_TPAD)
    lu_loss = -jnp.sum(du * (1.0 / SSL_TEMP) - jnp.log(ttl_u))
    li_loss = -jnp.sum(di * (1.0 / SSL_TEMP) - jnp.log(ttl_i))
    ssl = SSL_REG * (lu_loss + ALPHA * li_loss)
    ue = _norm_rows(lu_ref[...])
    pe = _norm_rows(lp_ref[...])
    ne = _norm_rows(ln_ref[...])
    x = jnp.sum(ue * pe, axis=1) - jnp.sum(ue * ne, axis=1)
    sig = 1.0 / (1.0 + jnp.exp(-x))
    mf = -jnp.mean(jnp.log(sig + 1e-6))
    reg = DECAY * 0.5 * (jnp.sum(u0 * u0) + jnp.sum(p0 * p0)
                         + jnp.sum(n0 * n0)) / BATCH
    mf_o[...] = jnp.reshape(mf, (1, 1))
    ssl_o[...] = jnp.reshape(ssl, (1, 1))
    reg_o[...] = jnp.reshape(reg, (1, 1))


def _sc_mesh():
    return plsc.VectorSubcoreMesh(core_axis_name="c", subcore_axis_name="s",
                                  num_cores=NC, num_subcores=NS)


_SC_PARAMS = pltpu.CompilerParams(use_tc_tiling_on_sc=False,
                                  needs_layout_passes=False)


def _make_layer_call():
    return pl.kernel(
        _layer_body,
        out_type=jax.ShapeDtypeStruct((N_NODES, EMB_DIM), jnp.float32),
        mesh=_sc_mesh(),
        compiler_params=_SC_PARAMS,
        scratch_types=[
            pltpu.VMEM((3, EDGE_ROW), jnp.int32),
            pltpu.VMEM((3, EDGE_ROW), jnp.int32),
            pltpu.VMEM((EDGE_ROW,), jnp.int32),
            pltpu.VMEM((EDGE_ROW,), jnp.int32),
            pltpu.VMEM((EDGE_ROW,), jnp.float32),
            pltpu.VMEM((EDGE_ROW,), jnp.float32),
            pltpu.VMEM((EDGE_ROW, EMB_DIM), jnp.float32),
            pltpu.VMEM((EDGE_ROW, EMB_DIM), jnp.float32),
            pltpu.VMEM((EDGE_ROW, EMB_DIM), jnp.float32),
            pltpu.VMEM((EDGE_ROW, EMB_DIM), jnp.float32),
            pltpu.VMEM_SHARED((ACC_ROWS, EMB_DIM), jnp.float32),
            pltpu.SemaphoreType.DMA,
            pltpu.SemaphoreType.DMA,
            pltpu.SemaphoreType.DMA,
            pltpu.SemaphoreType.DMA,
            pltpu.SemaphoreType.DMA,
            pltpu.SemaphoreType.DMA,
        ],
    )


def _make_gather_call():
    shp = jax.ShapeDtypeStruct((BATCH, EMB_DIM), jnp.float32)
    nb = BATCH // NW
    return pl.kernel(
        _gather_body,
        out_type=[shp] * 8,
        mesh=_sc_mesh(),
        compiler_params=_SC_PARAMS,
        scratch_types=[
            pltpu.VMEM((nb,), jnp.int32),
            pltpu.VMEM((nb,), jnp.int32),
            pltpu.VMEM((nb, EMB_DIM), jnp.float32),
            pltpu.VMEM((nb, EMB_DIM), jnp.float32),
        ],
    )


def kernel(embed_user, embed_item, edge_weight, edge_index, users, pos_items,
           neg_items, epoch):
    f32 = jnp.float32
    src = edge_index[0]
    dst = edge_index[1]
    pad = E_PAD - N_EDGES
    src2d = jnp.concatenate([src, jnp.zeros((pad,), src.dtype)]).reshape(-1, EDGE_ROW)
    dst2d = jnp.concatenate([dst, jnp.zeros((pad,), dst.dtype)]).reshape(-1, EDGE_ROW)
    wbits = lax.bitcast_convert_type(
        jnp.concatenate([edge_weight, jnp.zeros((pad,), f32)]), jnp.int32
    ).reshape(-1, EDGE_ROW)
    ed3 = jnp.stack([src2d, dst2d, wbits], axis=1)

    all0 = jnp.concatenate([embed_user, embed_item], axis=0)
    layer = _make_layer_call()
    e1 = layer(all0, ed3)
    e2 = layer(e1, ed3)
    e3 = layer(e2, ed3)

    gather = _make_gather_call()
    lsum_u, lsum_p, lsum_n, cu, ci, u0, p0, n0 = gather(
        embed_user, embed_item, e1, e2, e3, users, pos_items, neg_items)

    zpad = jnp.zeros((N_TPAD, EMB_DIM), f32)
    tabs = jnp.stack([jnp.concatenate([embed_user, zpad], axis=0),
                      jnp.concatenate([embed_item, zpad], axis=0)])
    cstack = jnp.stack([cu, ci])
    nblk = TAB_PAD // TBLK
    ttl_u, ttl_i = pl.pallas_call(
        _ttl_body,
        grid=(2, nblk),
        in_specs=[pl.BlockSpec((1, TBLK, EMB_DIM), lambda t, b: (t, b, 0)),
                  pl.BlockSpec((1, BATCH, EMB_DIM), lambda t, b: (t, 0, 0))],
        out_specs=[pl.BlockSpec((1, BATCH), lambda t, b: (0, 0))] * 2,
        out_shape=[jax.ShapeDtypeStruct((1, BATCH), f32)] * 2,
    )(tabs, cstack)
    ttl = jnp.concatenate([ttl_u, ttl_i], axis=0)

    mf, ssl, reg = pl.pallas_call(
        _final_body,
        out_shape=[jax.ShapeDtypeStruct((1, 1), f32)] * 3,
    )(ttl, cu, ci, u0, p0, n0, lsum_u, lsum_p, lsum_n)
    return (mf[0, 0], ssl[0, 0], reg[0, 0])
